# baseline (speedup 1.0000x reference)
import functools

import jax
import jax.numpy as jnp
from jax import lax
from jax.experimental import pallas as pl
from jax.experimental.pallas import tpu as pltpu
from jax.experimental.pallas import tpu_sc as plsc

N = 10000
E = 320000
D = 128
R = 16
LANES = 16
NC = 2
NS = 16
NW = NC * NS
CHUNK = 128
NCHUNKS = 2560
CPT = NCHUNKS // NW
E_PAD = NCHUNKS * CHUNK
DUMP_ROWS = 256
ROWS_PER_TILE = 624
ZROWS = 104
TAIL_ROWS = N - NS * ROWS_PER_TILE


def _zero_spmem(sid, agg_s, zbuf):
    rbase = sid * ROWS_PER_TILE
    for k in range(ROWS_PER_TILE // ZROWS):
        pltpu.sync_copy(zbuf, agg_s.at[pl.ds(rbase + k * ZROWS, ZROWS)])

    @pl.when(sid == 0)
    def _zero_tail():
        t0 = NS * ROWS_PER_TILE
        pltpu.sync_copy(zbuf.at[pl.ds(0, TAIL_ROWS)], agg_s.at[pl.ds(t0, TAIL_ROWS)])


def _export_spmem(cid, sid, agg_s, zbuf, out_hbm):
    rbase = sid * ROWS_PER_TILE
    for k in range(ROWS_PER_TILE // ZROWS):
        r0 = rbase + k * ZROWS
        pltpu.sync_copy(agg_s.at[pl.ds(r0, ZROWS)], zbuf)
        pltpu.sync_copy(zbuf, out_hbm.at[cid, pl.ds(r0, ZROWS)])

    @pl.when(sid == 0)
    def _export_tail():
        t0 = NS * ROWS_PER_TILE
        pltpu.sync_copy(agg_s.at[pl.ds(t0, TAIL_ROWS)], zbuf.at[pl.ds(0, TAIL_ROWS)])
        pltpu.sync_copy(zbuf.at[pl.ds(0, TAIL_ROWS)], out_hbm.at[cid, pl.ds(t0, TAIL_ROWS)])


def _sc_agg_body(compute_deg, gtab_hbm, fidx_hbm, dst_hbm, *refs):
    if compute_deg:
        (agg_out, deg_out, agg_s, fidx2, dst2, rows2, zbuf,
         gsem0, gsem1) = refs
    else:
        (agg_out, agg_s, fidx2, dst2, rows2, zbuf, gsem0, gsem1) = refs
    gsems = (gsem0, gsem1)
    cid = lax.axis_index("c")
    sid = lax.axis_index("s")
    wid = (1 - cid) * NS + sid
    cbase = wid * CPT

    def _zero_zbuf(i, _):
        for cb in range(D // LANES):
            zbuf[i, pl.ds(cb * LANES, LANES)] = jnp.zeros((LANES,), jnp.float32)
        return _
    lax.fori_loop(0, ZROWS, _zero_zbuf, None)

    _zero_spmem(sid, agg_s, zbuf)

    if compute_deg:
        def _ones(i, _):
            for cb in range(D // LANES):
                rows2[0, i, pl.ds(cb * LANES, LANES)] = jnp.ones((LANES,), jnp.float32)
            return _
        lax.fori_loop(0, CHUNK, _ones, None)
        plsc.subcore_barrier()

        def _deg_chunk(k, _):
            pltpu.sync_copy(dst_hbm.at[cbase + k], dst2.at[0])
            pltpu.sync_copy(rows2.at[0], agg_s.at[dst2.at[0]], add=True)
            return _
        lax.fori_loop(0, CPT, _deg_chunk, None)
        plsc.subcore_barrier()
        _export_spmem(cid, sid, agg_s, zbuf, deg_out)
        plsc.subcore_barrier()
        lax.fori_loop(0, ZROWS, _zero_zbuf, None)
        _zero_spmem(sid, agg_s, zbuf)

    plsc.subcore_barrier()

    for b in range(2):
        pltpu.sync_copy(fidx_hbm.at[cbase + b], fidx2.at[b])
        pltpu.sync_copy(dst_hbm.at[cbase + b], dst2.at[b])
        pltpu.async_copy(gtab_hbm.at[fidx2.at[b]], rows2.at[b], gsems[b])

    def _pipe(g2, _):
        k0 = g2 * 2
        for b in range(2):
            k = k0 + b
            pltpu.make_async_copy(gtab_hbm.at[fidx2.at[b]], rows2.at[b],
                                  gsems[b]).wait()
            pltpu.sync_copy(rows2.at[b], agg_s.at[dst2.at[b]], add=True)

            @pl.when(k + 2 < CPT)
            def _prefetch():
                pltpu.sync_copy(fidx_hbm.at[cbase + k + 2], fidx2.at[b])
                pltpu.sync_copy(dst_hbm.at[cbase + k + 2], dst2.at[b])
                pltpu.async_copy(gtab_hbm.at[fidx2.at[b]], rows2.at[b],
                                 gsems[b])
        return _
    lax.fori_loop(0, CPT // 2, _pipe, None)

    plsc.subcore_barrier()
    _export_spmem(cid, sid, agg_s, zbuf, agg_out)


def _make_sc_agg(compute_deg):
    mesh = plsc.VectorSubcoreMesh(core_axis_name="c", subcore_axis_name="s")
    agg_t = jax.ShapeDtypeStruct((NC, N, D), jnp.float32)
    out_type = (agg_t, agg_t) if compute_deg else agg_t
    return pl.kernel(
        functools.partial(_sc_agg_body, compute_deg),
        mesh=mesh,
        out_type=out_type,
        scratch_types=[
            pltpu.VMEM_SHARED((N + DUMP_ROWS, D), jnp.float32),
            pltpu.VMEM((2, CHUNK), jnp.int32),
            pltpu.VMEM((2, CHUNK), jnp.int32),
            pltpu.VMEM((2, CHUNK, D), jnp.float32),
            pltpu.VMEM((ZROWS, D), jnp.float32),
            pltpu.SemaphoreType.DMA,
            pltpu.SemaphoreType.DMA,
        ],
    )


def _pregate_body(h_ref, g_ref, o_ref):
    o_ref[...] = g_ref[...][:, None, :] * h_ref[...][None, :, :]


def _make_pregate():
    B = 1000
    return pl.pallas_call(
        _pregate_body,
        grid=(N // B,),
        in_specs=[
            pl.BlockSpec((B, D), lambda i: (i, 0)),
            pl.BlockSpec((R, D), lambda i: (0, 0)),
        ],
        out_specs=pl.BlockSpec((R, B, D), lambda i: (0, i, 0)),
        out_shape=jax.ShapeDtypeStruct((R, N, D), jnp.float32),
    )


def _fidx_body(src_ref, rel_ref, o_ref):
    o_ref[...] = rel_ref[...] * N + src_ref[...]


def _make_fidx():
    B = 256
    return pl.pallas_call(
        _fidx_body,
        grid=(NCHUNKS // B,),
        in_specs=[
            pl.BlockSpec((B, CHUNK), lambda i: (i, 0)),
            pl.BlockSpec((B, CHUNK), lambda i: (i, 0)),
        ],
        out_specs=pl.BlockSpec((B, CHUNK), lambda i: (i, 0)),
        out_shape=jax.ShapeDtypeStruct((NCHUNKS, CHUNK), jnp.int32),
    )


def _dense_body(final, agg_ref, deg_ref, h_ref, w_ref, b_ref, g_ref, bb_ref,
                ng_ref, nb_ref, o_ref):
    deg = deg_ref[0, :, 0:1] + deg_ref[1, :, 0:1]
    deg = jnp.maximum(deg, 1.0)
    agg = (agg_ref[0] + agg_ref[1]) / deg
    upd = jnp.dot(agg, w_ref[...], preferred_element_type=jnp.float32)
    upd = jnp.maximum(upd + b_ref[...], 0.0)
    x = h_ref[...] + upd
    mu = jnp.mean(x, axis=-1, keepdims=True)
    var = jnp.mean((x - mu) ** 2, axis=-1, keepdims=True)
    x = (x - mu) * lax.rsqrt(var + 1e-5) * g_ref[...] + bb_ref[...]
    if final:
        mu = jnp.mean(x, axis=-1, keepdims=True)
        var = jnp.mean((x - mu) ** 2, axis=-1, keepdims=True)
        x = (x - mu) * lax.rsqrt(var + 1e-5) * ng_ref[...] + nb_ref[...]
    o_ref[...] = x


def _make_dense(final):
    B = 1000
    return pl.pallas_call(
        functools.partial(_dense_body, final),
        grid=(N // B,),
        in_specs=[
            pl.BlockSpec((NC, B, D), lambda i: (0, i, 0)),
            pl.BlockSpec((NC, B, D), lambda i: (0, i, 0)),
            pl.BlockSpec((B, D), lambda i: (i, 0)),
            pl.BlockSpec((D, D), lambda i: (0, 0)),
            pl.BlockSpec((1, D), lambda i: (0, 0)),
            pl.BlockSpec((1, D), lambda i: (0, 0)),
            pl.BlockSpec((1, D), lambda i: (0, 0)),
            pl.BlockSpec((1, D), lambda i: (0, 0)),
            pl.BlockSpec((1, D), lambda i: (0, 0)),
        ],
        out_specs=pl.BlockSpec((B, D), lambda i: (i, 0)),
        out_shape=jax.ShapeDtypeStruct((N, D), jnp.float32),
    )


def kernel(V, edge_index, rel_ids, rel_gate, W_out, b_out, ln_g, ln_b,
           norm_g, norm_b):
    npad = E_PAD - E
    src = jnp.concatenate([edge_index[0], jnp.zeros((npad,), jnp.int32)])
    dst = jnp.concatenate(
        [edge_index[1], N + (jnp.arange(npad, dtype=jnp.int32) % DUMP_ROWS)])
    rel = jnp.concatenate([rel_ids, jnp.zeros((npad,), jnp.int32)])
    src = src.reshape(NCHUNKS, CHUNK)
    dst = dst.reshape(NCHUNKS, CHUNK)
    rel = rel.reshape(NCHUNKS, CHUNK)

    fidx = _make_fidx()(src, rel)
    pregate = _make_pregate()
    sc_agg0 = _make_sc_agg(True)
    sc_agg1 = _make_sc_agg(False)
    dense_mid = _make_dense(False)
    dense_fin = _make_dense(True)

    gtab = pregate(V, rel_gate[0]).reshape(R * N, D)
    agg2, deg = sc_agg0(gtab, fidx, dst)
    h = dense_mid(agg2, deg, V,
                  W_out[0], b_out[0][None, :], ln_g[0][None, :],
                  ln_b[0][None, :], norm_g[None, :], norm_b[None, :])
    gtab = pregate(h, rel_gate[1]).reshape(R * N, D)
    agg2 = sc_agg1(gtab, fidx, dst)
    h = dense_fin(agg2, deg, h,
                  W_out[1], b_out[1][None, :], ln_g[1][None, :],
                  ln_b[1][None, :], norm_g[None, :], norm_b[None, :])
    return h


# speedup vs baseline: 1.2043x; 1.2043x over previous
import functools

import jax
import jax.numpy as jnp
from jax import lax
from jax.experimental import pallas as pl
from jax.experimental.pallas import tpu as pltpu
from jax.experimental.pallas import tpu_sc as plsc

N = 10000
E = 320000
D = 128
R = 16
LANES = 16
NC = 2
NS = 16
NW = NC * NS
CHUNK = 128
NCHUNKS = 2560
CPT = NCHUNKS // NW
E_PAD = NCHUNKS * CHUNK
DUMP_ROWS = 256
ROWS_PER_TILE = 624
ZROWS = 104
TAIL_ROWS = N - NS * ROWS_PER_TILE


def _zero_spmem(sid, agg_s, zbuf):
    rbase = sid * ROWS_PER_TILE
    for k in range(ROWS_PER_TILE // ZROWS):
        pltpu.sync_copy(zbuf, agg_s.at[pl.ds(rbase + k * ZROWS, ZROWS)])

    @pl.when(sid == 0)
    def _zero_tail():
        t0 = NS * ROWS_PER_TILE
        pltpu.sync_copy(zbuf.at[pl.ds(0, TAIL_ROWS)], agg_s.at[pl.ds(t0, TAIL_ROWS)])


def _export_spmem(cid, sid, agg_s, zbuf, out_hbm):
    rbase = sid * ROWS_PER_TILE
    for k in range(ROWS_PER_TILE // ZROWS):
        r0 = rbase + k * ZROWS
        pltpu.sync_copy(agg_s.at[pl.ds(r0, ZROWS)], zbuf)
        pltpu.sync_copy(zbuf, out_hbm.at[cid, pl.ds(r0, ZROWS)])

    @pl.when(sid == 0)
    def _export_tail():
        t0 = NS * ROWS_PER_TILE
        pltpu.sync_copy(agg_s.at[pl.ds(t0, TAIL_ROWS)], zbuf.at[pl.ds(0, TAIL_ROWS)])
        pltpu.sync_copy(zbuf.at[pl.ds(0, TAIL_ROWS)], out_hbm.at[cid, pl.ds(t0, TAIL_ROWS)])


def _sc_agg_body(compute_deg, gtab_hbm, fidx_hbm, dst_hbm, *refs):
    if compute_deg:
        (agg_out, deg_out, agg_s, fidx2, dst2, rows2, zbuf,
         gsem0, gsem1) = refs
    else:
        (agg_out, agg_s, fidx2, dst2, rows2, zbuf, gsem0, gsem1) = refs
    gsems = (gsem0, gsem1)
    cid = lax.axis_index("c")
    sid = lax.axis_index("s")
    wid = cid * NS + sid

    def _zero_zbuf(i, _):
        for cb in range(D // LANES):
            zbuf[i, pl.ds(cb * LANES, LANES)] = jnp.zeros((LANES,), jnp.float32)
        return _
    lax.fori_loop(0, ZROWS, _zero_zbuf, None)

    _zero_spmem(sid, agg_s, zbuf)

    if compute_deg:
        def _ones(i, _):
            for cb in range(D // LANES):
                rows2[0, i, pl.ds(cb * LANES, LANES)] = jnp.ones((LANES,), jnp.float32)
            return _
        lax.fori_loop(0, CHUNK, _ones, None)
        plsc.subcore_barrier()

        def _deg_chunk(k, _):
            pltpu.sync_copy(dst_hbm.at[wid + k * NW], dst2.at[0])
            pltpu.sync_copy(rows2.at[0], agg_s.at[dst2.at[0]], add=True)
            return _
        lax.fori_loop(0, CPT, _deg_chunk, None)
        plsc.subcore_barrier()
        _export_spmem(cid, sid, agg_s, zbuf, deg_out)
        plsc.subcore_barrier()
        lax.fori_loop(0, ZROWS, _zero_zbuf, None)
        _zero_spmem(sid, agg_s, zbuf)

    plsc.subcore_barrier()

    for b in range(2):
        pltpu.sync_copy(fidx_hbm.at[wid + b * NW], fidx2.at[b])
        pltpu.sync_copy(dst_hbm.at[wid + b * NW], dst2.at[b])
        pltpu.async_copy(gtab_hbm.at[fidx2.at[b]], rows2.at[b], gsems[b])

    def _pipe(g2, _):
        k0 = g2 * 2
        for b in range(2):
            k = k0 + b
            pltpu.make_async_copy(gtab_hbm.at[fidx2.at[b]], rows2.at[b],
                                  gsems[b]).wait()
            pltpu.sync_copy(rows2.at[b], agg_s.at[dst2.at[b]], add=True)

            @pl.when(k + 2 < CPT)
            def _prefetch():
                pltpu.sync_copy(fidx_hbm.at[wid + (k + 2) * NW], fidx2.at[b])
                pltpu.sync_copy(dst_hbm.at[wid + (k + 2) * NW], dst2.at[b])
                pltpu.async_copy(gtab_hbm.at[fidx2.at[b]], rows2.at[b],
                                 gsems[b])
        return _
    lax.fori_loop(0, CPT // 2, _pipe, None)

    plsc.subcore_barrier()
    _export_spmem(cid, sid, agg_s, zbuf, agg_out)


def _make_sc_agg(compute_deg):
    mesh = plsc.VectorSubcoreMesh(core_axis_name="c", subcore_axis_name="s")
    agg_t = jax.ShapeDtypeStruct((NC, N, D), jnp.float32)
    out_type = (agg_t, agg_t) if compute_deg else agg_t
    return pl.kernel(
        functools.partial(_sc_agg_body, compute_deg),
        mesh=mesh,
        out_type=out_type,
        scratch_types=[
            pltpu.VMEM_SHARED((N + DUMP_ROWS, D), jnp.float32),
            pltpu.VMEM((2, CHUNK), jnp.int32),
            pltpu.VMEM((2, CHUNK), jnp.int32),
            pltpu.VMEM((2, CHUNK, D), jnp.float32),
            pltpu.VMEM((ZROWS, D), jnp.float32),
            pltpu.SemaphoreType.DMA,
            pltpu.SemaphoreType.DMA,
        ],
    )


def _pregate_body(h_ref, g_ref, o_ref):
    o_ref[...] = g_ref[...][:, None, :] * h_ref[...][None, :, :]


def _make_pregate():
    B = 1000
    return pl.pallas_call(
        _pregate_body,
        grid=(N // B,),
        in_specs=[
            pl.BlockSpec((B, D), lambda i: (i, 0)),
            pl.BlockSpec((R, D), lambda i: (0, 0)),
        ],
        out_specs=pl.BlockSpec((R, B, D), lambda i: (0, i, 0)),
        out_shape=jax.ShapeDtypeStruct((R, N, D), jnp.float32),
    )


def _fidx_body(src_ref, rel_ref, o_ref):
    o_ref[...] = rel_ref[...] * N + src_ref[...]


def _make_fidx():
    B = 256
    return pl.pallas_call(
        _fidx_body,
        grid=(NCHUNKS // B,),
        in_specs=[
            pl.BlockSpec((B, CHUNK), lambda i: (i, 0)),
            pl.BlockSpec((B, CHUNK), lambda i: (i, 0)),
        ],
        out_specs=pl.BlockSpec((B, CHUNK), lambda i: (i, 0)),
        out_shape=jax.ShapeDtypeStruct((NCHUNKS, CHUNK), jnp.int32),
    )


def _dense_body(final, agg_ref, deg_ref, h_ref, w_ref, b_ref, g_ref, bb_ref,
                ng_ref, nb_ref, o_ref):
    deg = deg_ref[0, :, 0:1] + deg_ref[1, :, 0:1]
    deg = jnp.maximum(deg, 1.0)
    agg = (agg_ref[0] + agg_ref[1]) / deg
    upd = jnp.dot(agg, w_ref[...], preferred_element_type=jnp.float32)
    upd = jnp.maximum(upd + b_ref[...], 0.0)
    x = h_ref[...] + upd
    mu = jnp.mean(x, axis=-1, keepdims=True)
    var = jnp.mean((x - mu) ** 2, axis=-1, keepdims=True)
    x = (x - mu) * lax.rsqrt(var + 1e-5) * g_ref[...] + bb_ref[...]
    if final:
        mu = jnp.mean(x, axis=-1, keepdims=True)
        var = jnp.mean((x - mu) ** 2, axis=-1, keepdims=True)
        x = (x - mu) * lax.rsqrt(var + 1e-5) * ng_ref[...] + nb_ref[...]
    o_ref[...] = x


def _make_dense(final):
    B = 1000
    return pl.pallas_call(
        functools.partial(_dense_body, final),
        grid=(N // B,),
        in_specs=[
            pl.BlockSpec((NC, B, D), lambda i: (0, i, 0)),
            pl.BlockSpec((NC, B, D), lambda i: (0, i, 0)),
            pl.BlockSpec((B, D), lambda i: (i, 0)),
            pl.BlockSpec((D, D), lambda i: (0, 0)),
            pl.BlockSpec((1, D), lambda i: (0, 0)),
            pl.BlockSpec((1, D), lambda i: (0, 0)),
            pl.BlockSpec((1, D), lambda i: (0, 0)),
            pl.BlockSpec((1, D), lambda i: (0, 0)),
            pl.BlockSpec((1, D), lambda i: (0, 0)),
        ],
        out_specs=pl.BlockSpec((B, D), lambda i: (i, 0)),
        out_shape=jax.ShapeDtypeStruct((N, D), jnp.float32),
    )


def kernel(V, edge_index, rel_ids, rel_gate, W_out, b_out, ln_g, ln_b,
           norm_g, norm_b):
    npad = E_PAD - E
    src = jnp.concatenate([edge_index[0], jnp.zeros((npad,), jnp.int32)])
    dst = jnp.concatenate(
        [edge_index[1], N + (jnp.arange(npad, dtype=jnp.int32) % DUMP_ROWS)])
    rel = jnp.concatenate([rel_ids, jnp.zeros((npad,), jnp.int32)])
    src = src.reshape(NCHUNKS, CHUNK)
    dst = dst.reshape(NCHUNKS, CHUNK)
    rel = rel.reshape(NCHUNKS, CHUNK)

    fidx = _make_fidx()(src, rel)
    pregate = _make_pregate()
    sc_agg0 = _make_sc_agg(True)
    sc_agg1 = _make_sc_agg(False)
    dense_mid = _make_dense(False)
    dense_fin = _make_dense(True)

    gtab = pregate(V, rel_gate[0]).reshape(R * N, D)
    agg2, deg = sc_agg0(gtab, fidx, dst)
    h = dense_mid(agg2, deg, V,
                  W_out[0], b_out[0][None, :], ln_g[0][None, :],
                  ln_b[0][None, :], norm_g[None, :], norm_b[None, :])
    gtab = pregate(h, rel_gate[1]).reshape(R * N, D)
    agg2 = sc_agg1(gtab, fidx, dst)
    h = dense_fin(agg2, deg, h,
                  W_out[1], b_out[1][None, :], ln_g[1][None, :],
                  ln_b[1][None, :], norm_g[None, :], norm_b[None, :])
    return h


# speedup vs baseline: 1.2425x; 1.0317x over previous
import functools

import jax
import jax.numpy as jnp
from jax import lax
from jax.experimental import pallas as pl
from jax.experimental.pallas import tpu as pltpu
from jax.experimental.pallas import tpu_sc as plsc

N = 10000
E = 320000
D = 128
R = 16
LANES = 16
NC = 2
NS = 16
NW = NC * NS
CHUNK = 128
NCHUNKS = 2560
CPT = NCHUNKS // NW
E_PAD = NCHUNKS * CHUNK
DUMP_ROWS = 256
ROWS_PER_TILE = 624
ZROWS = 104
TAIL_ROWS = N - NS * ROWS_PER_TILE


def _zero_spmem(sid, agg_s, zbuf):
    rbase = sid * ROWS_PER_TILE
    for k in range(ROWS_PER_TILE // ZROWS):
        pltpu.sync_copy(zbuf, agg_s.at[pl.ds(rbase + k * ZROWS, ZROWS)])

    @pl.when(sid == 0)
    def _zero_tail():
        t0 = NS * ROWS_PER_TILE
        pltpu.sync_copy(zbuf.at[pl.ds(0, TAIL_ROWS)], agg_s.at[pl.ds(t0, TAIL_ROWS)])


def _export_spmem(cid, sid, agg_s, zbuf, out_hbm):
    rbase = sid * ROWS_PER_TILE
    for k in range(ROWS_PER_TILE // ZROWS):
        r0 = rbase + k * ZROWS
        pltpu.sync_copy(agg_s.at[pl.ds(r0, ZROWS)], zbuf)
        pltpu.sync_copy(zbuf, out_hbm.at[cid, pl.ds(r0, ZROWS)])

    @pl.when(sid == 0)
    def _export_tail():
        t0 = NS * ROWS_PER_TILE
        pltpu.sync_copy(agg_s.at[pl.ds(t0, TAIL_ROWS)], zbuf.at[pl.ds(0, TAIL_ROWS)])
        pltpu.sync_copy(zbuf.at[pl.ds(0, TAIL_ROWS)], out_hbm.at[cid, pl.ds(t0, TAIL_ROWS)])


def _sc_agg_body(compute_deg, gtab_hbm, fd_hbm, *refs):
    if compute_deg:
        (agg_out, deg_out, agg_s, fd2, rows2, zbuf,
         gsem0, gsem1, isem0, isem1) = refs
    else:
        (agg_out, agg_s, fd2, rows2, zbuf, gsem0, gsem1, isem0, isem1) = refs
    gsems = (gsem0, gsem1)
    isems = (isem0, isem1)
    cid = lax.axis_index("c")
    sid = lax.axis_index("s")
    wid = cid * NS + sid

    def _zero_zbuf(i, _):
        for cb in range(D // LANES):
            zbuf[i, pl.ds(cb * LANES, LANES)] = jnp.zeros((LANES,), jnp.float32)
        return _
    lax.fori_loop(0, ZROWS, _zero_zbuf, None)

    _zero_spmem(sid, agg_s, zbuf)

    if compute_deg:
        def _ones(i, _):
            for cb in range(D // LANES):
                rows2[0, i, pl.ds(cb * LANES, LANES)] = jnp.ones((LANES,), jnp.float32)
            return _
        lax.fori_loop(0, CHUNK, _ones, None)
        plsc.subcore_barrier()

        def _deg_chunk(k, _):
            pltpu.sync_copy(fd_hbm.at[wid + k * NW], fd2.at[0])
            pltpu.sync_copy(rows2.at[0], agg_s.at[fd2.at[0, 1]], add=True)
            return _
        lax.fori_loop(0, CPT, _deg_chunk, None)
        plsc.subcore_barrier()
        _export_spmem(cid, sid, agg_s, zbuf, deg_out)
        plsc.subcore_barrier()
        lax.fori_loop(0, ZROWS, _zero_zbuf, None)
        _zero_spmem(sid, agg_s, zbuf)

    plsc.subcore_barrier()

    pltpu.sync_copy(fd_hbm.at[wid], fd2.at[0])
    pltpu.async_copy(fd_hbm.at[wid + NW], fd2.at[1], isems[1])
    pltpu.async_copy(gtab_hbm.at[fd2.at[0, 0]], rows2.at[0], gsems[0])

    def _pipe(g2, _):
        k0 = g2 * 2
        for b in range(2):
            k = k0 + b
            b1 = 1 - b
            pltpu.make_async_copy(gtab_hbm.at[fd2.at[b, 0]], rows2.at[b],
                                  gsems[b]).wait()

            @pl.when(k + 1 < CPT)
            def _next_gather():
                pltpu.make_async_copy(fd_hbm.at[wid], fd2.at[b1],
                                      isems[b1]).wait()
                pltpu.async_copy(gtab_hbm.at[fd2.at[b1, 0]], rows2.at[b1],
                                 gsems[b1])

            pltpu.sync_copy(rows2.at[b], agg_s.at[fd2.at[b, 1]], add=True)

            @pl.when(k + 2 < CPT)
            def _prefetch_idx():
                pltpu.async_copy(fd_hbm.at[wid + (k + 2) * NW], fd2.at[b],
                                 isems[b])
        return _
    lax.fori_loop(0, CPT // 2, _pipe, None)

    plsc.subcore_barrier()
    _export_spmem(cid, sid, agg_s, zbuf, agg_out)


def _make_sc_agg(compute_deg):
    mesh = plsc.VectorSubcoreMesh(core_axis_name="c", subcore_axis_name="s")
    agg_t = jax.ShapeDtypeStruct((NC, N, D), jnp.float32)
    out_type = (agg_t, agg_t) if compute_deg else agg_t
    return pl.kernel(
        functools.partial(_sc_agg_body, compute_deg),
        mesh=mesh,
        out_type=out_type,
        scratch_types=[
            pltpu.VMEM_SHARED((N + DUMP_ROWS, D), jnp.float32),
            pltpu.VMEM((2, 2, CHUNK), jnp.int32),
            pltpu.VMEM((2, CHUNK, D), jnp.float32),
            pltpu.VMEM((ZROWS, D), jnp.float32),
            pltpu.SemaphoreType.DMA,
            pltpu.SemaphoreType.DMA,
            pltpu.SemaphoreType.DMA,
            pltpu.SemaphoreType.DMA,
        ],
    )


def _pregate_body(h_ref, g_ref, o_ref):
    o_ref[...] = g_ref[...][:, None, :] * h_ref[...][None, :, :]


def _make_pregate():
    B = 1000
    return pl.pallas_call(
        _pregate_body,
        grid=(N // B,),
        in_specs=[
            pl.BlockSpec((B, D), lambda i: (i, 0)),
            pl.BlockSpec((R, D), lambda i: (0, 0)),
        ],
        out_specs=pl.BlockSpec((R, B, D), lambda i: (0, i, 0)),
        out_shape=jax.ShapeDtypeStruct((R, N, D), jnp.float32),
    )


def _fidx_body(src_ref, rel_ref, o_ref):
    o_ref[...] = rel_ref[...] * N + src_ref[...]


def _make_fidx():
    B = 256
    return pl.pallas_call(
        _fidx_body,
        grid=(NCHUNKS // B,),
        in_specs=[
            pl.BlockSpec((B, CHUNK), lambda i: (i, 0)),
            pl.BlockSpec((B, CHUNK), lambda i: (i, 0)),
        ],
        out_specs=pl.BlockSpec((B, CHUNK), lambda i: (i, 0)),
        out_shape=jax.ShapeDtypeStruct((NCHUNKS, CHUNK), jnp.int32),
    )


def _dense_body(final, agg_ref, deg_ref, h_ref, w_ref, b_ref, g_ref, bb_ref,
                ng_ref, nb_ref, o_ref):
    deg = deg_ref[0, :, 0:1] + deg_ref[1, :, 0:1]
    deg = jnp.maximum(deg, 1.0)
    agg = (agg_ref[0] + agg_ref[1]) / deg
    upd = jnp.dot(agg, w_ref[...], preferred_element_type=jnp.float32)
    upd = jnp.maximum(upd + b_ref[...], 0.0)
    x = h_ref[...] + upd
    mu = jnp.mean(x, axis=-1, keepdims=True)
    var = jnp.mean((x - mu) ** 2, axis=-1, keepdims=True)
    x = (x - mu) * lax.rsqrt(var + 1e-5) * g_ref[...] + bb_ref[...]
    if final:
        mu = jnp.mean(x, axis=-1, keepdims=True)
        var = jnp.mean((x - mu) ** 2, axis=-1, keepdims=True)
        x = (x - mu) * lax.rsqrt(var + 1e-5) * ng_ref[...] + nb_ref[...]
    o_ref[...] = x


def _make_dense(final):
    B = 1000
    return pl.pallas_call(
        functools.partial(_dense_body, final),
        grid=(N // B,),
        in_specs=[
            pl.BlockSpec((NC, B, D), lambda i: (0, i, 0)),
            pl.BlockSpec((NC, B, D), lambda i: (0, i, 0)),
            pl.BlockSpec((B, D), lambda i: (i, 0)),
            pl.BlockSpec((D, D), lambda i: (0, 0)),
            pl.BlockSpec((1, D), lambda i: (0, 0)),
            pl.BlockSpec((1, D), lambda i: (0, 0)),
            pl.BlockSpec((1, D), lambda i: (0, 0)),
            pl.BlockSpec((1, D), lambda i: (0, 0)),
            pl.BlockSpec((1, D), lambda i: (0, 0)),
        ],
        out_specs=pl.BlockSpec((B, D), lambda i: (i, 0)),
        out_shape=jax.ShapeDtypeStruct((N, D), jnp.float32),
    )


def kernel(V, edge_index, rel_ids, rel_gate, W_out, b_out, ln_g, ln_b,
           norm_g, norm_b):
    npad = E_PAD - E
    src = jnp.concatenate([edge_index[0], jnp.zeros((npad,), jnp.int32)])
    dst = jnp.concatenate(
        [edge_index[1], N + (jnp.arange(npad, dtype=jnp.int32) % DUMP_ROWS)])
    rel = jnp.concatenate([rel_ids, jnp.zeros((npad,), jnp.int32)])
    src = src.reshape(NCHUNKS, CHUNK)
    dst = dst.reshape(NCHUNKS, CHUNK)
    rel = rel.reshape(NCHUNKS, CHUNK)

    fidx = _make_fidx()(src, rel)
    fd = jnp.stack([fidx, dst], axis=1)
    pregate = _make_pregate()
    sc_agg0 = _make_sc_agg(True)
    sc_agg1 = _make_sc_agg(False)
    dense_mid = _make_dense(False)
    dense_fin = _make_dense(True)

    gtab = pregate(V, rel_gate[0]).reshape(R * N, D)
    agg2, deg = sc_agg0(gtab, fd)
    h = dense_mid(agg2, deg, V,
                  W_out[0], b_out[0][None, :], ln_g[0][None, :],
                  ln_b[0][None, :], norm_g[None, :], norm_b[None, :])
    gtab = pregate(h, rel_gate[1]).reshape(R * N, D)
    agg2 = sc_agg1(gtab, fd)
    h = dense_fin(agg2, deg, h,
                  W_out[1], b_out[1][None, :], ln_g[1][None, :],
                  ln_b[1][None, :], norm_g[None, :], norm_b[None, :])
    return h


# speedup vs baseline: 2.6933x; 2.1677x over previous
import functools

import jax
import jax.numpy as jnp
from jax import lax
from jax.experimental import pallas as pl
from jax.experimental.pallas import tpu as pltpu
from jax.experimental.pallas import tpu_sc as plsc

N = 10000
E = 320000
D = 128
R = 16
LANES = 16
NC = 2
NS = 16
NW = NC * NS
CHUNK = 128
NCHUNKS = 2560
CPT = NCHUNKS // NW
E_PAD = NCHUNKS * CHUNK
DUMP_ROWS = 256
ROWS_PER_TILE = 624
ZROWS = 104
TAIL_ROWS = N - NS * ROWS_PER_TILE


def _zero_spmem(sid, agg_s, zbuf):
    rbase = sid * ROWS_PER_TILE
    for k in range(ROWS_PER_TILE // ZROWS):
        pltpu.sync_copy(zbuf, agg_s.at[pl.ds(rbase + k * ZROWS, ZROWS)])

    @pl.when(sid == 0)
    def _zero_tail():
        t0 = NS * ROWS_PER_TILE
        pltpu.sync_copy(zbuf.at[pl.ds(0, TAIL_ROWS)], agg_s.at[pl.ds(t0, TAIL_ROWS)])


def _export_spmem(cid, sid, agg_s, zbuf, out_hbm):
    rbase = sid * ROWS_PER_TILE
    for k in range(ROWS_PER_TILE // ZROWS):
        r0 = rbase + k * ZROWS
        pltpu.sync_copy(agg_s.at[pl.ds(r0, ZROWS)], zbuf)
        pltpu.sync_copy(zbuf, out_hbm.at[cid, pl.ds(r0, ZROWS)])

    @pl.when(sid == 0)
    def _export_tail():
        t0 = NS * ROWS_PER_TILE
        pltpu.sync_copy(agg_s.at[pl.ds(t0, TAIL_ROWS)], zbuf.at[pl.ds(0, TAIL_ROWS)])
        pltpu.sync_copy(zbuf.at[pl.ds(0, TAIL_ROWS)], out_hbm.at[cid, pl.ds(t0, TAIL_ROWS)])


def _sc_agg_body(compute_deg, gtab_hbm, fd_hbm, *refs):
    if compute_deg:
        (agg_out, deg_out, agg_s, fd2, rows2, zbuf,
         gsem0, gsem1, isem0, isem1) = refs
    else:
        (agg_out, agg_s, fd2, rows2, zbuf, gsem0, gsem1, isem0, isem1) = refs
    gsems = (gsem0, gsem1)
    isems = (isem0, isem1)
    cid = lax.axis_index("c")
    sid = lax.axis_index("s")
    wid = cid * NS + sid

    def _zero_zbuf(i, _):
        for cb in range(D // LANES):
            zbuf[i, pl.ds(cb * LANES, LANES)] = jnp.zeros((LANES,), jnp.float32)
        return _
    lax.fori_loop(0, ZROWS, _zero_zbuf, None)

    _zero_spmem(sid, agg_s, zbuf)

    if compute_deg:
        def _ones(i, _):
            for cb in range(D // LANES):
                rows2[0, i, pl.ds(cb * LANES, LANES)] = jnp.ones((LANES,), jnp.float32)
            return _
        lax.fori_loop(0, CHUNK, _ones, None)
        plsc.subcore_barrier()

        def _deg_chunk(k, _):
            pltpu.sync_copy(fd_hbm.at[wid + k * NW], fd2.at[0])
            pltpu.sync_copy(rows2.at[0], agg_s.at[fd2.at[0, 1]], add=True)
            return _
        lax.fori_loop(0, CPT, _deg_chunk, None)
        plsc.subcore_barrier()
        _export_spmem(cid, sid, agg_s, zbuf, deg_out)
        plsc.subcore_barrier()
        lax.fori_loop(0, ZROWS, _zero_zbuf, None)
        _zero_spmem(sid, agg_s, zbuf)

    plsc.subcore_barrier()

    pltpu.sync_copy(fd_hbm.at[wid], fd2.at[0])
    pltpu.async_copy(fd_hbm.at[wid + NW], fd2.at[1], isems[1])
    pltpu.async_copy(gtab_hbm.at[fd2.at[0, 0]], rows2.at[0], gsems[0])

    def _pipe(g2, _):
        k0 = g2 * 2
        for b in range(2):
            k = k0 + b
            b1 = 1 - b
            pltpu.make_async_copy(gtab_hbm.at[fd2.at[b, 0]], rows2.at[b],
                                  gsems[b]).wait()

            @pl.when(k + 1 < CPT)
            def _next_gather():
                pltpu.make_async_copy(fd_hbm.at[wid], fd2.at[b1],
                                      isems[b1]).wait()
                pltpu.async_copy(gtab_hbm.at[fd2.at[b1, 0]], rows2.at[b1],
                                 gsems[b1])

            pltpu.sync_copy(rows2.at[b], agg_s.at[fd2.at[b, 1]], add=True)

            @pl.when(k + 2 < CPT)
            def _prefetch_idx():
                pltpu.async_copy(fd_hbm.at[wid + (k + 2) * NW], fd2.at[b],
                                 isems[b])
        return _
    lax.fori_loop(0, CPT // 2, _pipe, None)

    plsc.subcore_barrier()
    _export_spmem(cid, sid, agg_s, zbuf, agg_out)


def _make_sc_agg(compute_deg):
    mesh = plsc.VectorSubcoreMesh(core_axis_name="c", subcore_axis_name="s")
    agg_t = jax.ShapeDtypeStruct((NC, N, D), jnp.float32)
    out_type = (agg_t, agg_t) if compute_deg else agg_t
    return pl.kernel(
        functools.partial(_sc_agg_body, compute_deg),
        mesh=mesh,
        out_type=out_type,
        scratch_types=[
            pltpu.VMEM_SHARED((N + DUMP_ROWS, D), jnp.float32),
            pltpu.VMEM((2, 2, CHUNK), jnp.int32),
            pltpu.VMEM((2, CHUNK, D), jnp.float32),
            pltpu.VMEM((ZROWS, D), jnp.float32),
            pltpu.SemaphoreType.DMA,
            pltpu.SemaphoreType.DMA,
            pltpu.SemaphoreType.DMA,
            pltpu.SemaphoreType.DMA,
        ],
    )


def _pregate_body(h_ref, g_ref, o_ref):
    o_ref[...] = g_ref[...][:, None, :] * h_ref[...][None, :, :]


def _make_pregate():
    B = 1000
    return pl.pallas_call(
        _pregate_body,
        grid=(N // B,),
        in_specs=[
            pl.BlockSpec((B, D), lambda i: (i, 0)),
            pl.BlockSpec((R, D), lambda i: (0, 0)),
        ],
        out_specs=pl.BlockSpec((R, B, D), lambda i: (0, i, 0)),
        out_shape=jax.ShapeDtypeStruct((R, N, D), jnp.float32),
    )


def _fidx_body(src_ref, rel_ref, o_ref):
    o_ref[...] = rel_ref[...] * N + src_ref[...]


def _make_fidx():
    B = 256
    return pl.pallas_call(
        _fidx_body,
        grid=(NCHUNKS // B,),
        in_specs=[
            pl.BlockSpec((B, CHUNK), lambda i: (i, 0)),
            pl.BlockSpec((B, CHUNK), lambda i: (i, 0)),
        ],
        out_specs=pl.BlockSpec((B, CHUNK), lambda i: (i, 0)),
        out_shape=jax.ShapeDtypeStruct((NCHUNKS, CHUNK), jnp.int32),
    )


def _dense_body(final, agg_ref, deg_ref, h_ref, w_ref, b_ref, g_ref, bb_ref,
                ng_ref, nb_ref, o_ref):
    deg = deg_ref[0, :, 0:1] + deg_ref[1, :, 0:1]
    deg = jnp.maximum(deg, 1.0)
    agg = (agg_ref[0] + agg_ref[1]) / deg
    upd = jnp.dot(agg, w_ref[...], preferred_element_type=jnp.float32)
    upd = jnp.maximum(upd + b_ref[...], 0.0)
    x = h_ref[...] + upd
    mu = jnp.mean(x, axis=-1, keepdims=True)
    var = jnp.mean((x - mu) ** 2, axis=-1, keepdims=True)
    x = (x - mu) * lax.rsqrt(var + 1e-5) * g_ref[...] + bb_ref[...]
    if final:
        mu = jnp.mean(x, axis=-1, keepdims=True)
        var = jnp.mean((x - mu) ** 2, axis=-1, keepdims=True)
        x = (x - mu) * lax.rsqrt(var + 1e-5) * ng_ref[...] + nb_ref[...]
    o_ref[...] = x


def _make_dense(final):
    B = 1000
    return pl.pallas_call(
        functools.partial(_dense_body, final),
        grid=(N // B,),
        in_specs=[
            pl.BlockSpec((NC, B, D), lambda i: (0, i, 0)),
            pl.BlockSpec((NC, B, D), lambda i: (0, i, 0)),
            pl.BlockSpec((B, D), lambda i: (i, 0)),
            pl.BlockSpec((D, D), lambda i: (0, 0)),
            pl.BlockSpec((1, D), lambda i: (0, 0)),
            pl.BlockSpec((1, D), lambda i: (0, 0)),
            pl.BlockSpec((1, D), lambda i: (0, 0)),
            pl.BlockSpec((1, D), lambda i: (0, 0)),
            pl.BlockSpec((1, D), lambda i: (0, 0)),
        ],
        out_specs=pl.BlockSpec((B, D), lambda i: (i, 0)),
        out_shape=jax.ShapeDtypeStruct((N, D), jnp.float32),
    )


def kernel(V, edge_index, rel_ids, rel_gate, W_out, b_out, ln_g, ln_b,
           norm_g, norm_b):
    npad = E_PAD - E
    ppos = jnp.arange(npad, dtype=jnp.int32)
    src = jnp.concatenate([edge_index[0], (ppos * 37) % N])
    dst = jnp.concatenate([edge_index[1], N + (ppos % DUMP_ROWS)])
    rel = jnp.concatenate([rel_ids, ppos % R])
    src = src.reshape(NCHUNKS, CHUNK)
    dst = dst.reshape(NCHUNKS, CHUNK)
    rel = rel.reshape(NCHUNKS, CHUNK)

    fidx = _make_fidx()(src, rel)
    fd = jnp.stack([fidx, dst], axis=1)
    pregate = _make_pregate()
    sc_agg0 = _make_sc_agg(True)
    sc_agg1 = _make_sc_agg(False)
    dense_mid = _make_dense(False)
    dense_fin = _make_dense(True)

    gtab = pregate(V, rel_gate[0]).reshape(R * N, D)
    agg2, deg = sc_agg0(gtab, fd)
    h = dense_mid(agg2, deg, V,
                  W_out[0], b_out[0][None, :], ln_g[0][None, :],
                  ln_b[0][None, :], norm_g[None, :], norm_b[None, :])
    gtab = pregate(h, rel_gate[1]).reshape(R * N, D)
    agg2 = sc_agg1(gtab, fd)
    h = dense_fin(agg2, deg, h,
                  W_out[1], b_out[1][None, :], ln_g[1][None, :],
                  ln_b[1][None, :], norm_g[None, :], norm_b[None, :])
    return h


# speedup vs baseline: 3.1112x; 1.1552x over previous
import functools

import jax
import jax.numpy as jnp
from jax import lax
from jax.experimental import pallas as pl
from jax.experimental.pallas import tpu as pltpu
from jax.experimental.pallas import tpu_sc as plsc

N = 10000
E = 320000
D = 128
R = 16
LANES = 16
NC = 2
NS = 16
NW = NC * NS
CHUNK = 128
NCHUNKS = 2560
CPT = NCHUNKS // NW
E_PAD = NCHUNKS * CHUNK
DUMP_ROWS = 256
ROWS_PER_TILE = 624
ZROWS = 104
TAIL_ROWS = N - NS * ROWS_PER_TILE


def _zero_spmem(sid, agg_s, zbuf):
    rbase = sid * ROWS_PER_TILE
    for k in range(ROWS_PER_TILE // ZROWS):
        pltpu.sync_copy(zbuf, agg_s.at[pl.ds(rbase + k * ZROWS, ZROWS)])

    @pl.when(sid == 0)
    def _zero_tail():
        t0 = NS * ROWS_PER_TILE
        pltpu.sync_copy(zbuf.at[pl.ds(0, TAIL_ROWS)], agg_s.at[pl.ds(t0, TAIL_ROWS)])


def _export_spmem(cid, sid, agg_s, zbuf, out_hbm):
    rbase = sid * ROWS_PER_TILE
    for k in range(ROWS_PER_TILE // ZROWS):
        r0 = rbase + k * ZROWS
        pltpu.sync_copy(agg_s.at[pl.ds(r0, ZROWS)], zbuf)
        pltpu.sync_copy(zbuf, out_hbm.at[cid, pl.ds(r0, ZROWS)])

    @pl.when(sid == 0)
    def _export_tail():
        t0 = NS * ROWS_PER_TILE
        pltpu.sync_copy(agg_s.at[pl.ds(t0, TAIL_ROWS)], zbuf.at[pl.ds(0, TAIL_ROWS)])
        pltpu.sync_copy(zbuf.at[pl.ds(0, TAIL_ROWS)], out_hbm.at[cid, pl.ds(t0, TAIL_ROWS)])


def _zero_zbuf_loop(zbuf):
    def _zero_zbuf(i, _):
        for cb in range(D // LANES):
            zbuf[i, pl.ds(cb * LANES, LANES)] = jnp.zeros((LANES,), jnp.float32)
        return _
    lax.fori_loop(0, ZROWS, _zero_zbuf, None)


def _sc_agg_body(gtab_hbm, fd_hbm, agg_out, agg_s, fd2, rows2, zbuf,
                 gsem0, gsem1, isem0, isem1):
    gsems = (gsem0, gsem1)
    isems = (isem0, isem1)
    cid = lax.axis_index("c")
    sid = lax.axis_index("s")
    wid = cid * NS + sid

    _zero_zbuf_loop(zbuf)
    _zero_spmem(sid, agg_s, zbuf)
    plsc.subcore_barrier()

    pltpu.sync_copy(fd_hbm.at[wid], fd2.at[0])
    pltpu.async_copy(fd_hbm.at[wid + NW], fd2.at[1], isems[1])
    pltpu.async_copy(gtab_hbm.at[fd2.at[0, 0]], rows2.at[0], gsems[0])

    def _pipe(g2, _):
        k0 = g2 * 2
        for b in range(2):
            k = k0 + b
            b1 = 1 - b
            pltpu.make_async_copy(gtab_hbm.at[fd2.at[b, 0]], rows2.at[b],
                                  gsems[b]).wait()

            @pl.when(k + 1 < CPT)
            def _next_gather():
                pltpu.make_async_copy(fd_hbm.at[wid], fd2.at[b1],
                                      isems[b1]).wait()
                pltpu.async_copy(gtab_hbm.at[fd2.at[b1, 0]], rows2.at[b1],
                                 gsems[b1])

            pltpu.sync_copy(rows2.at[b], agg_s.at[fd2.at[b, 1]], add=True)

            @pl.when(k + 2 < CPT)
            def _prefetch_idx():
                pltpu.async_copy(fd_hbm.at[wid + (k + 2) * NW], fd2.at[b],
                                 isems[b])
        return _
    lax.fori_loop(0, CPT // 2, _pipe, None)

    plsc.subcore_barrier()
    _export_spmem(cid, sid, agg_s, zbuf, agg_out)


def _sc_deg_body(fd_hbm, deg_out, agg_s, fd2, rows2, zbuf, isem0, isem1):
    isems = (isem0, isem1)
    cid = lax.axis_index("c")
    sid = lax.axis_index("s")
    wid = cid * NS + sid

    _zero_zbuf_loop(zbuf)
    _zero_spmem(sid, agg_s, zbuf)

    def _ones(i, _):
        for cb in range(D // LANES):
            rows2[0, i, pl.ds(cb * LANES, LANES)] = jnp.ones((LANES,), jnp.float32)
        return _
    lax.fori_loop(0, CHUNK, _ones, None)
    plsc.subcore_barrier()

    pltpu.sync_copy(fd_hbm.at[wid], fd2.at[0])
    pltpu.sync_copy(fd_hbm.at[wid + NW], fd2.at[1])

    def _pipe(g2, _):
        k0 = g2 * 2
        for b in range(2):
            k = k0 + b

            @pl.when(k >= 2)
            def _wait_idx():
                pltpu.make_async_copy(fd_hbm.at[wid], fd2.at[b],
                                      isems[b]).wait()

            pltpu.sync_copy(rows2.at[0], agg_s.at[fd2.at[b, 1]], add=True)

            @pl.when(k + 2 < CPT)
            def _prefetch_idx():
                pltpu.async_copy(fd_hbm.at[wid + (k + 2) * NW], fd2.at[b],
                                 isems[b])
        return _
    lax.fori_loop(0, CPT // 2, _pipe, None)

    plsc.subcore_barrier()
    _export_spmem(cid, sid, agg_s, zbuf, deg_out)


def _make_sc(body, n_out, n_sems):
    mesh = plsc.VectorSubcoreMesh(core_axis_name="c", subcore_axis_name="s")
    agg_t = jax.ShapeDtypeStruct((NC, N, D), jnp.float32)
    return pl.kernel(
        body,
        mesh=mesh,
        out_type=agg_t,
        scratch_types=[
            pltpu.VMEM_SHARED((N + DUMP_ROWS, D), jnp.float32),
            pltpu.VMEM((2, 2, CHUNK), jnp.int32),
            pltpu.VMEM((2, CHUNK, D), jnp.float32),
            pltpu.VMEM((ZROWS, D), jnp.float32),
        ] + [pltpu.SemaphoreType.DMA] * n_sems,
    )


def _fd_body(src_ref, rel_ref, dst_ref, o_ref):
    o_ref[:, 0, :] = rel_ref[...] * N + src_ref[...]
    o_ref[:, 1, :] = dst_ref[...]


def _make_fd():
    B = 256
    return pl.pallas_call(
        _fd_body,
        grid=(NCHUNKS // B,),
        in_specs=[
            pl.BlockSpec((B, CHUNK), lambda i: (i, 0)),
            pl.BlockSpec((B, CHUNK), lambda i: (i, 0)),
            pl.BlockSpec((B, CHUNK), lambda i: (i, 0)),
        ],
        out_specs=pl.BlockSpec((B, 2, CHUNK), lambda i: (i, 0, 0)),
        out_shape=jax.ShapeDtypeStruct((NCHUNKS, 2, CHUNK), jnp.int32),
    )


def _pregate_body(h_ref, g_ref, o_ref):
    o_ref[...] = g_ref[...][:, None, :] * h_ref[...][None, :, :]


def _make_pregate():
    B = 1000
    return pl.pallas_call(
        _pregate_body,
        grid=(N // B,),
        in_specs=[
            pl.BlockSpec((B, D), lambda i: (i, 0)),
            pl.BlockSpec((R, D), lambda i: (0, 0)),
        ],
        out_specs=pl.BlockSpec((R, B, D), lambda i: (0, i, 0)),
        out_shape=jax.ShapeDtypeStruct((R, N, D), jnp.float32),
    )


def _dense_body(final, agg_ref, deg_ref, h_ref, w_ref, b_ref, g_ref, bb_ref,
                ng_ref, nb_ref, *out_refs):
    deg = deg_ref[0, :, 0:1] + deg_ref[1, :, 0:1]
    deg = jnp.maximum(deg, 1.0)
    agg = (agg_ref[0] + agg_ref[1]) / deg
    upd = jnp.dot(agg, w_ref[...], preferred_element_type=jnp.float32)
    upd = jnp.maximum(upd + b_ref[...], 0.0)
    x = h_ref[...] + upd
    mu = jnp.mean(x, axis=-1, keepdims=True)
    var = jnp.mean((x - mu) ** 2, axis=-1, keepdims=True)
    x = (x - mu) * lax.rsqrt(var + 1e-5) * g_ref[...] + bb_ref[...]
    if final:
        mu = jnp.mean(x, axis=-1, keepdims=True)
        var = jnp.mean((x - mu) ** 2, axis=-1, keepdims=True)
        x = (x - mu) * lax.rsqrt(var + 1e-5) * ng_ref[...] + nb_ref[...]
        out_refs[0][...] = x
    else:
        out_refs[0][...] = x
        out_refs[1][...] = ng_ref[...][:, None, :] * x[None, :, :]


def _make_dense(final):
    B = 1000
    if final:
        out_specs = pl.BlockSpec((B, D), lambda i: (i, 0))
        out_shape = jax.ShapeDtypeStruct((N, D), jnp.float32)
        gate_spec = pl.BlockSpec((1, D), lambda i: (0, 0))
    else:
        out_specs = (pl.BlockSpec((B, D), lambda i: (i, 0)),
                     pl.BlockSpec((R, B, D), lambda i: (0, i, 0)))
        out_shape = (jax.ShapeDtypeStruct((N, D), jnp.float32),
                     jax.ShapeDtypeStruct((R, N, D), jnp.float32))
        gate_spec = pl.BlockSpec((R, D), lambda i: (0, 0))
    return pl.pallas_call(
        functools.partial(_dense_body, final),
        grid=(N // B,),
        in_specs=[
            pl.BlockSpec((NC, B, D), lambda i: (0, i, 0)),
            pl.BlockSpec((NC, B, D), lambda i: (0, i, 0)),
            pl.BlockSpec((B, D), lambda i: (i, 0)),
            pl.BlockSpec((D, D), lambda i: (0, 0)),
            pl.BlockSpec((1, D), lambda i: (0, 0)),
            pl.BlockSpec((1, D), lambda i: (0, 0)),
            pl.BlockSpec((1, D), lambda i: (0, 0)),
            gate_spec,
            pl.BlockSpec((1, D), lambda i: (0, 0)),
        ],
        out_specs=out_specs,
        out_shape=out_shape,
    )


def kernel(V, edge_index, rel_ids, rel_gate, W_out, b_out, ln_g, ln_b,
           norm_g, norm_b):
    npad = E_PAD - E
    ppos = jnp.arange(npad, dtype=jnp.int32)
    src = jnp.concatenate([edge_index[0], (ppos * 37) % N])
    dst = jnp.concatenate([edge_index[1], N + (ppos % DUMP_ROWS)])
    rel = jnp.concatenate([rel_ids, ppos % R])
    src = src.reshape(NCHUNKS, CHUNK)
    dst = dst.reshape(NCHUNKS, CHUNK)
    rel = rel.reshape(NCHUNKS, CHUNK)

    fd = _make_fd()(src, rel, dst)
    sc_agg = _make_sc(_sc_agg_body, 1, 4)
    sc_deg = _make_sc(_sc_deg_body, 1, 2)
    dense_mid = _make_dense(False)
    dense_fin = _make_dense(True)

    deg = sc_deg(fd)
    gtab = _make_pregate()(V, rel_gate[0]).reshape(R * N, D)
    agg2 = sc_agg(gtab, fd)
    h, gtab = dense_mid(agg2, deg, V,
                        W_out[0], b_out[0][None, :], ln_g[0][None, :],
                        ln_b[0][None, :], rel_gate[1], norm_b[None, :])
    agg2 = sc_agg(gtab.reshape(R * N, D), fd)
    h = dense_fin(agg2, deg, h,
                  W_out[1], b_out[1][None, :], ln_g[1][None, :],
                  ln_b[1][None, :], norm_g[None, :], norm_b[None, :])
    return h


# speedup vs baseline: 3.1188x; 1.0024x over previous
import functools

import jax
import jax.numpy as jnp
from jax import lax
from jax.experimental import pallas as pl
from jax.experimental.pallas import tpu as pltpu
from jax.experimental.pallas import tpu_sc as plsc

N = 10000
E = 320000
D = 128
R = 16
LANES = 16
NC = 2
NS = 16
NW = NC * NS
CHUNK = 128
NCHUNKS = 2560
CPT = NCHUNKS // NW
E_PAD = NCHUNKS * CHUNK
DUMP_ROWS = 256
ROWS_PER_TILE = 624
ZROWS = 104
TAIL_ROWS = N - NS * ROWS_PER_TILE


def _zero_spmem(sid, agg_s, zbuf):
    rbase = sid * ROWS_PER_TILE
    for k in range(ROWS_PER_TILE // ZROWS):
        pltpu.sync_copy(zbuf, agg_s.at[pl.ds(rbase + k * ZROWS, ZROWS)])

    @pl.when(sid == 0)
    def _zero_tail():
        t0 = NS * ROWS_PER_TILE
        pltpu.sync_copy(zbuf.at[pl.ds(0, TAIL_ROWS)], agg_s.at[pl.ds(t0, TAIL_ROWS)])


def _export_spmem(cid, sid, agg_s, zbuf, out_hbm):
    rbase = sid * ROWS_PER_TILE
    for k in range(ROWS_PER_TILE // ZROWS):
        r0 = rbase + k * ZROWS
        pltpu.sync_copy(agg_s.at[pl.ds(r0, ZROWS)], zbuf)
        pltpu.sync_copy(zbuf, out_hbm.at[cid, pl.ds(r0, ZROWS)])

    @pl.when(sid == 0)
    def _export_tail():
        t0 = NS * ROWS_PER_TILE
        pltpu.sync_copy(agg_s.at[pl.ds(t0, TAIL_ROWS)], zbuf.at[pl.ds(0, TAIL_ROWS)])
        pltpu.sync_copy(zbuf.at[pl.ds(0, TAIL_ROWS)], out_hbm.at[cid, pl.ds(t0, TAIL_ROWS)])


def _zero_zbuf_loop(zbuf):
    def _zero_zbuf(i, _):
        for cb in range(D // LANES):
            zbuf[i, pl.ds(cb * LANES, LANES)] = jnp.zeros((LANES,), jnp.float32)
        return _
    lax.fori_loop(0, ZROWS, _zero_zbuf, None)


def _sc_agg_body(gtab_hbm, fd_hbm, agg_out, agg_s, fd4, rows2, zbuf,
                 gsem0, gsem1, isem0, isem1, ssem0, ssem1):
    gsems = (gsem0, gsem1)
    isems = (isem0, isem1)
    ssems = (ssem0, ssem1)
    cid = lax.axis_index("c")
    sid = lax.axis_index("s")
    wid = cid * NS + sid

    _zero_zbuf_loop(zbuf)
    _zero_spmem(sid, agg_s, zbuf)
    plsc.subcore_barrier()

    pltpu.sync_copy(fd_hbm.at[wid], fd4.at[0])
    pltpu.async_copy(fd_hbm.at[wid + NW], fd4.at[1], isems[1])
    pltpu.async_copy(gtab_hbm.at[fd4.at[0, 0]], rows2.at[0], gsems[0])

    def _pipe(g2, _):
        k0 = g2 * 2
        for b in range(2):
            k = k0 + b
            b1 = 1 - b
            f = k % 4
            f1 = (k + 1) % 4
            pltpu.make_async_copy(gtab_hbm.at[fd4.at[f, 0]], rows2.at[b],
                                  gsems[b]).wait()

            @pl.when(k + 1 < CPT)
            def _next_gather():
                pltpu.make_async_copy(fd_hbm.at[wid], fd4.at[f1],
                                      isems[b1]).wait()

                @pl.when(k >= 1)
                def _free_rows():
                    pltpu.make_async_copy(
                        rows2.at[b1], agg_s.at[fd4.at[(k + 3) % 4, 1]],
                        ssems[b1]).wait()

                pltpu.async_copy(gtab_hbm.at[fd4.at[f1, 0]], rows2.at[b1],
                                 gsems[b1])

            pltpu.async_copy(rows2.at[b], agg_s.at[fd4.at[f, 1]], ssems[b],
                             add=True)

            @pl.when(k + 2 < CPT)
            def _prefetch_idx():
                pltpu.async_copy(fd_hbm.at[wid + (k + 2) * NW],
                                 fd4.at[(k + 2) % 4], isems[b])
        return _
    lax.fori_loop(0, CPT // 2, _pipe, None)

    for b in range(2):
        pltpu.make_async_copy(rows2.at[b], agg_s.at[fd4.at[b, 1]],
                              ssems[b]).wait()

    plsc.subcore_barrier()
    _export_spmem(cid, sid, agg_s, zbuf, agg_out)


def _sc_deg_body(fd_hbm, deg_out, agg_s, fd2, rows2, zbuf, isem0, isem1):
    isems = (isem0, isem1)
    cid = lax.axis_index("c")
    sid = lax.axis_index("s")
    wid = cid * NS + sid

    _zero_zbuf_loop(zbuf)
    _zero_spmem(sid, agg_s, zbuf)

    def _ones(i, _):
        for cb in range(D // LANES):
            rows2[0, i, pl.ds(cb * LANES, LANES)] = jnp.ones((LANES,), jnp.float32)
        return _
    lax.fori_loop(0, CHUNK, _ones, None)
    plsc.subcore_barrier()

    pltpu.sync_copy(fd_hbm.at[wid], fd2.at[0])
    pltpu.sync_copy(fd_hbm.at[wid + NW], fd2.at[1])

    def _pipe(g2, _):
        k0 = g2 * 2
        for b in range(2):
            k = k0 + b

            @pl.when(k >= 2)
            def _wait_idx():
                pltpu.make_async_copy(fd_hbm.at[wid], fd2.at[b],
                                      isems[b]).wait()

            pltpu.sync_copy(rows2.at[0], agg_s.at[fd2.at[b, 1]], add=True)

            @pl.when(k + 2 < CPT)
            def _prefetch_idx():
                pltpu.async_copy(fd_hbm.at[wid + (k + 2) * NW], fd2.at[b],
                                 isems[b])
        return _
    lax.fori_loop(0, CPT // 2, _pipe, None)

    plsc.subcore_barrier()
    _export_spmem(cid, sid, agg_s, zbuf, deg_out)


def _make_sc(body, fd_depth, n_sems):
    mesh = plsc.VectorSubcoreMesh(core_axis_name="c", subcore_axis_name="s")
    agg_t = jax.ShapeDtypeStruct((NC, N, D), jnp.float32)
    return pl.kernel(
        body,
        mesh=mesh,
        out_type=agg_t,
        scratch_types=[
            pltpu.VMEM_SHARED((N + DUMP_ROWS, D), jnp.float32),
            pltpu.VMEM((fd_depth, 2, CHUNK), jnp.int32),
            pltpu.VMEM((2, CHUNK, D), jnp.float32),
            pltpu.VMEM((ZROWS, D), jnp.float32),
        ] + [pltpu.SemaphoreType.DMA] * n_sems,
    )


def _fd_body(src_ref, rel_ref, dst_ref, o_ref):
    o_ref[:, 0, :] = rel_ref[...] * N + src_ref[...]
    o_ref[:, 1, :] = dst_ref[...]


def _make_fd():
    B = 256
    return pl.pallas_call(
        _fd_body,
        grid=(NCHUNKS // B,),
        in_specs=[
            pl.BlockSpec((B, CHUNK), lambda i: (i, 0)),
            pl.BlockSpec((B, CHUNK), lambda i: (i, 0)),
            pl.BlockSpec((B, CHUNK), lambda i: (i, 0)),
        ],
        out_specs=pl.BlockSpec((B, 2, CHUNK), lambda i: (i, 0, 0)),
        out_shape=jax.ShapeDtypeStruct((NCHUNKS, 2, CHUNK), jnp.int32),
    )


def _pregate_body(h_ref, g_ref, o_ref):
    o_ref[...] = g_ref[...][:, None, :] * h_ref[...][None, :, :]


def _make_pregate():
    B = 1000
    return pl.pallas_call(
        _pregate_body,
        grid=(N // B,),
        in_specs=[
            pl.BlockSpec((B, D), lambda i: (i, 0)),
            pl.BlockSpec((R, D), lambda i: (0, 0)),
        ],
        out_specs=pl.BlockSpec((R, B, D), lambda i: (0, i, 0)),
        out_shape=jax.ShapeDtypeStruct((R, N, D), jnp.float32),
    )


def _dense_body(final, agg_ref, deg_ref, h_ref, w_ref, b_ref, g_ref, bb_ref,
                ng_ref, nb_ref, *out_refs):
    deg = deg_ref[0, :, 0:1] + deg_ref[1, :, 0:1]
    deg = jnp.maximum(deg, 1.0)
    agg = (agg_ref[0] + agg_ref[1]) / deg
    upd = jnp.dot(agg, w_ref[...], preferred_element_type=jnp.float32)
    upd = jnp.maximum(upd + b_ref[...], 0.0)
    x = h_ref[...] + upd
    mu = jnp.mean(x, axis=-1, keepdims=True)
    var = jnp.mean((x - mu) ** 2, axis=-1, keepdims=True)
    x = (x - mu) * lax.rsqrt(var + 1e-5) * g_ref[...] + bb_ref[...]
    if final:
        mu = jnp.mean(x, axis=-1, keepdims=True)
        var = jnp.mean((x - mu) ** 2, axis=-1, keepdims=True)
        x = (x - mu) * lax.rsqrt(var + 1e-5) * ng_ref[...] + nb_ref[...]
        out_refs[0][...] = x
    else:
        out_refs[0][...] = x
        out_refs[1][...] = ng_ref[...][:, None, :] * x[None, :, :]


def _make_dense(final):
    B = 1000
    if final:
        out_specs = pl.BlockSpec((B, D), lambda i: (i, 0))
        out_shape = jax.ShapeDtypeStruct((N, D), jnp.float32)
        gate_spec = pl.BlockSpec((1, D), lambda i: (0, 0))
    else:
        out_specs = (pl.BlockSpec((B, D), lambda i: (i, 0)),
                     pl.BlockSpec((R, B, D), lambda i: (0, i, 0)))
        out_shape = (jax.ShapeDtypeStruct((N, D), jnp.float32),
                     jax.ShapeDtypeStruct((R, N, D), jnp.float32))
        gate_spec = pl.BlockSpec((R, D), lambda i: (0, 0))
    return pl.pallas_call(
        functools.partial(_dense_body, final),
        grid=(N // B,),
        in_specs=[
            pl.BlockSpec((NC, B, D), lambda i: (0, i, 0)),
            pl.BlockSpec((NC, B, D), lambda i: (0, i, 0)),
            pl.BlockSpec((B, D), lambda i: (i, 0)),
            pl.BlockSpec((D, D), lambda i: (0, 0)),
            pl.BlockSpec((1, D), lambda i: (0, 0)),
            pl.BlockSpec((1, D), lambda i: (0, 0)),
            pl.BlockSpec((1, D), lambda i: (0, 0)),
            gate_spec,
            pl.BlockSpec((1, D), lambda i: (0, 0)),
        ],
        out_specs=out_specs,
        out_shape=out_shape,
    )


def kernel(V, edge_index, rel_ids, rel_gate, W_out, b_out, ln_g, ln_b,
           norm_g, norm_b):
    npad = E_PAD - E
    ppos = jnp.arange(npad, dtype=jnp.int32)
    src = jnp.concatenate([edge_index[0], (ppos * 37) % N])
    dst = jnp.concatenate([edge_index[1], N + (ppos % DUMP_ROWS)])
    rel = jnp.concatenate([rel_ids, ppos % R])
    src = src.reshape(NCHUNKS, CHUNK)
    dst = dst.reshape(NCHUNKS, CHUNK)
    rel = rel.reshape(NCHUNKS, CHUNK)

    fd = _make_fd()(src, rel, dst)
    sc_agg = _make_sc(_sc_agg_body, 4, 6)
    sc_deg = _make_sc(_sc_deg_body, 2, 2)
    dense_mid = _make_dense(False)
    dense_fin = _make_dense(True)

    deg = sc_deg(fd)
    gtab = _make_pregate()(V, rel_gate[0]).reshape(R * N, D)
    agg2 = sc_agg(gtab, fd)
    h, gtab = dense_mid(agg2, deg, V,
                        W_out[0], b_out[0][None, :], ln_g[0][None, :],
                        ln_b[0][None, :], rel_gate[1], norm_b[None, :])
    agg2 = sc_agg(gtab.reshape(R * N, D), fd)
    h = dense_fin(agg2, deg, h,
                  W_out[1], b_out[1][None, :], ln_g[1][None, :],
                  ln_b[1][None, :], norm_g[None, :], norm_b[None, :])
    return h


# speedup vs baseline: 3.1202x; 1.0005x over previous
import functools

import jax
import jax.numpy as jnp
from jax import lax
from jax.experimental import pallas as pl
from jax.experimental.pallas import tpu as pltpu
from jax.experimental.pallas import tpu_sc as plsc

N = 10000
E = 320000
D = 128
R = 16
LANES = 16
NC = 2
NS = 16
NW = NC * NS
CHUNK = 128
NCHUNKS = 2560
CPT = NCHUNKS // NW
E_PAD = NCHUNKS * CHUNK
DUMP_ROWS = 256
ROWS_PER_TILE = 624
ZROWS = 104
TAIL_ROWS = N - NS * ROWS_PER_TILE


def _zero_spmem(sid, agg_s, zbuf):
    rbase = sid * ROWS_PER_TILE
    for k in range(ROWS_PER_TILE // ZROWS):
        pltpu.sync_copy(zbuf, agg_s.at[pl.ds(rbase + k * ZROWS, ZROWS)])

    @pl.when(sid == 0)
    def _zero_tail():
        t0 = NS * ROWS_PER_TILE
        pltpu.sync_copy(zbuf.at[pl.ds(0, TAIL_ROWS)], agg_s.at[pl.ds(t0, TAIL_ROWS)])


def _export_spmem(cid, sid, agg_s, zbuf, out_hbm):
    rbase = sid * ROWS_PER_TILE
    pltpu.sync_copy(agg_s.at[pl.ds(rbase, ROWS_PER_TILE)],
                    out_hbm.at[cid, pl.ds(rbase, ROWS_PER_TILE)])

    @pl.when(sid == 0)
    def _export_tail():
        t0 = NS * ROWS_PER_TILE
        pltpu.sync_copy(agg_s.at[pl.ds(t0, TAIL_ROWS)],
                        out_hbm.at[cid, pl.ds(t0, TAIL_ROWS)])


def _zero_zbuf_loop(zbuf):
    def _zero_zbuf(i, _):
        for cb in range(D // LANES):
            zbuf[i, pl.ds(cb * LANES, LANES)] = jnp.zeros((LANES,), jnp.float32)
        return _
    lax.fori_loop(0, ZROWS, _zero_zbuf, None)


def _sc_agg_body(gtab_hbm, fd_hbm, agg_out, agg_s, fd4, rows2, zbuf,
                 gsem0, gsem1, isem0, isem1, ssem0, ssem1):
    gsems = (gsem0, gsem1)
    isems = (isem0, isem1)
    ssems = (ssem0, ssem1)
    cid = lax.axis_index("c")
    sid = lax.axis_index("s")
    wid = cid * NS + sid

    _zero_zbuf_loop(zbuf)
    _zero_spmem(sid, agg_s, zbuf)
    plsc.subcore_barrier()

    pltpu.sync_copy(fd_hbm.at[wid], fd4.at[0])
    pltpu.async_copy(fd_hbm.at[wid + NW], fd4.at[1], isems[1])
    pltpu.async_copy(gtab_hbm.at[fd4.at[0, 0]], rows2.at[0], gsems[0])

    def _pipe(g2, _):
        k0 = g2 * 2
        for b in range(2):
            k = k0 + b
            b1 = 1 - b
            f = k % 4
            f1 = (k + 1) % 4
            pltpu.make_async_copy(gtab_hbm.at[fd4.at[f, 0]], rows2.at[b],
                                  gsems[b]).wait()

            @pl.when(k + 1 < CPT)
            def _next_gather():
                pltpu.make_async_copy(fd_hbm.at[wid], fd4.at[f1],
                                      isems[b1]).wait()

                @pl.when(k >= 1)
                def _free_rows():
                    pltpu.make_async_copy(
                        rows2.at[b1], agg_s.at[fd4.at[(k + 3) % 4, 1]],
                        ssems[b1]).wait()

                pltpu.async_copy(gtab_hbm.at[fd4.at[f1, 0]], rows2.at[b1],
                                 gsems[b1])

            pltpu.async_copy(rows2.at[b], agg_s.at[fd4.at[f, 1]], ssems[b],
                             add=True)

            @pl.when(k + 2 < CPT)
            def _prefetch_idx():
                pltpu.async_copy(fd_hbm.at[wid + (k + 2) * NW],
                                 fd4.at[(k + 2) % 4], isems[b])
        return _
    lax.fori_loop(0, CPT // 2, _pipe, None)

    for b in range(2):
        pltpu.make_async_copy(rows2.at[b], agg_s.at[fd4.at[b, 1]],
                              ssems[b]).wait()

    plsc.subcore_barrier()
    _export_spmem(cid, sid, agg_s, zbuf, agg_out)


def _sc_deg_body(fd_hbm, deg_out, agg_s, fd2, rows2, zbuf, isem0, isem1):
    isems = (isem0, isem1)
    cid = lax.axis_index("c")
    sid = lax.axis_index("s")
    wid = cid * NS + sid

    _zero_zbuf_loop(zbuf)
    _zero_spmem(sid, agg_s, zbuf)

    def _ones(i, _):
        for cb in range(D // LANES):
            rows2[0, i, pl.ds(cb * LANES, LANES)] = jnp.ones((LANES,), jnp.float32)
        return _
    lax.fori_loop(0, CHUNK, _ones, None)
    plsc.subcore_barrier()

    pltpu.sync_copy(fd_hbm.at[wid], fd2.at[0])
    pltpu.sync_copy(fd_hbm.at[wid + NW], fd2.at[1])

    def _pipe(g2, _):
        k0 = g2 * 2
        for b in range(2):
            k = k0 + b

            @pl.when(k >= 2)
            def _wait_idx():
                pltpu.make_async_copy(fd_hbm.at[wid], fd2.at[b],
                                      isems[b]).wait()

            pltpu.sync_copy(rows2.at[0], agg_s.at[fd2.at[b, 1]], add=True)

            @pl.when(k + 2 < CPT)
            def _prefetch_idx():
                pltpu.async_copy(fd_hbm.at[wid + (k + 2) * NW], fd2.at[b],
                                 isems[b])
        return _
    lax.fori_loop(0, CPT // 2, _pipe, None)

    plsc.subcore_barrier()
    _export_spmem(cid, sid, agg_s, zbuf, deg_out)


def _make_sc(body, fd_depth, n_sems):
    mesh = plsc.VectorSubcoreMesh(core_axis_name="c", subcore_axis_name="s")
    agg_t = jax.ShapeDtypeStruct((NC, N, D), jnp.float32)
    return pl.kernel(
        body,
        mesh=mesh,
        out_type=agg_t,
        scratch_types=[
            pltpu.VMEM_SHARED((N + DUMP_ROWS, D), jnp.float32),
            pltpu.VMEM((fd_depth, 2, CHUNK), jnp.int32),
            pltpu.VMEM((2, CHUNK, D), jnp.float32),
            pltpu.VMEM((ZROWS, D), jnp.float32),
        ] + [pltpu.SemaphoreType.DMA] * n_sems,
    )


def _fd_body(src_ref, rel_ref, dst_ref, o_ref):
    o_ref[:, 0, :] = rel_ref[...] * N + src_ref[...]
    o_ref[:, 1, :] = dst_ref[...]


def _make_fd():
    B = 256
    return pl.pallas_call(
        _fd_body,
        grid=(NCHUNKS // B,),
        in_specs=[
            pl.BlockSpec((B, CHUNK), lambda i: (i, 0)),
            pl.BlockSpec((B, CHUNK), lambda i: (i, 0)),
            pl.BlockSpec((B, CHUNK), lambda i: (i, 0)),
        ],
        out_specs=pl.BlockSpec((B, 2, CHUNK), lambda i: (i, 0, 0)),
        out_shape=jax.ShapeDtypeStruct((NCHUNKS, 2, CHUNK), jnp.int32),
    )


def _pregate_body(h_ref, g_ref, o_ref):
    o_ref[...] = g_ref[...][:, None, :] * h_ref[...][None, :, :]


def _make_pregate():
    B = 1000
    return pl.pallas_call(
        _pregate_body,
        grid=(N // B,),
        in_specs=[
            pl.BlockSpec((B, D), lambda i: (i, 0)),
            pl.BlockSpec((R, D), lambda i: (0, 0)),
        ],
        out_specs=pl.BlockSpec((R, B, D), lambda i: (0, i, 0)),
        out_shape=jax.ShapeDtypeStruct((R, N, D), jnp.float32),
    )


def _dense_body(final, agg_ref, deg_ref, h_ref, w_ref, b_ref, g_ref, bb_ref,
                ng_ref, nb_ref, *out_refs):
    deg = deg_ref[0, :, 0:1] + deg_ref[1, :, 0:1]
    deg = jnp.maximum(deg, 1.0)
    agg = (agg_ref[0] + agg_ref[1]) / deg
    upd = jnp.dot(agg, w_ref[...], preferred_element_type=jnp.float32)
    upd = jnp.maximum(upd + b_ref[...], 0.0)
    x = h_ref[...] + upd
    mu = jnp.mean(x, axis=-1, keepdims=True)
    var = jnp.mean((x - mu) ** 2, axis=-1, keepdims=True)
    x = (x - mu) * lax.rsqrt(var + 1e-5) * g_ref[...] + bb_ref[...]
    if final:
        mu = jnp.mean(x, axis=-1, keepdims=True)
        var = jnp.mean((x - mu) ** 2, axis=-1, keepdims=True)
        x = (x - mu) * lax.rsqrt(var + 1e-5) * ng_ref[...] + nb_ref[...]
        out_refs[0][...] = x
    else:
        out_refs[0][...] = x
        out_refs[1][...] = ng_ref[...][:, None, :] * x[None, :, :]


def _make_dense(final):
    B = 1000
    if final:
        out_specs = pl.BlockSpec((B, D), lambda i: (i, 0))
        out_shape = jax.ShapeDtypeStruct((N, D), jnp.float32)
        gate_spec = pl.BlockSpec((1, D), lambda i: (0, 0))
    else:
        out_specs = (pl.BlockSpec((B, D), lambda i: (i, 0)),
                     pl.BlockSpec((R, B, D), lambda i: (0, i, 0)))
        out_shape = (jax.ShapeDtypeStruct((N, D), jnp.float32),
                     jax.ShapeDtypeStruct((R, N, D), jnp.float32))
        gate_spec = pl.BlockSpec((R, D), lambda i: (0, 0))
    return pl.pallas_call(
        functools.partial(_dense_body, final),
        grid=(N // B,),
        in_specs=[
            pl.BlockSpec((NC, B, D), lambda i: (0, i, 0)),
            pl.BlockSpec((NC, B, D), lambda i: (0, i, 0)),
            pl.BlockSpec((B, D), lambda i: (i, 0)),
            pl.BlockSpec((D, D), lambda i: (0, 0)),
            pl.BlockSpec((1, D), lambda i: (0, 0)),
            pl.BlockSpec((1, D), lambda i: (0, 0)),
            pl.BlockSpec((1, D), lambda i: (0, 0)),
            gate_spec,
            pl.BlockSpec((1, D), lambda i: (0, 0)),
        ],
        out_specs=out_specs,
        out_shape=out_shape,
    )


def kernel(V, edge_index, rel_ids, rel_gate, W_out, b_out, ln_g, ln_b,
           norm_g, norm_b):
    npad = E_PAD - E
    ppos = jnp.arange(npad, dtype=jnp.int32)
    src = jnp.concatenate([edge_index[0], (ppos * 37) % N])
    dst = jnp.concatenate([edge_index[1], N + (ppos % DUMP_ROWS)])
    rel = jnp.concatenate([rel_ids, ppos % R])
    src = src.reshape(NCHUNKS, CHUNK)
    dst = dst.reshape(NCHUNKS, CHUNK)
    rel = rel.reshape(NCHUNKS, CHUNK)

    fd = _make_fd()(src, rel, dst)
    sc_agg = _make_sc(_sc_agg_body, 4, 6)
    sc_deg = _make_sc(_sc_deg_body, 2, 2)
    dense_mid = _make_dense(False)
    dense_fin = _make_dense(True)

    deg = sc_deg(fd)
    gtab = _make_pregate()(V, rel_gate[0]).reshape(R * N, D)
    agg2 = sc_agg(gtab, fd)
    h, gtab = dense_mid(agg2, deg, V,
                        W_out[0], b_out[0][None, :], ln_g[0][None, :],
                        ln_b[0][None, :], rel_gate[1], norm_b[None, :])
    agg2 = sc_agg(gtab.reshape(R * N, D), fd)
    h = dense_fin(agg2, deg, h,
                  W_out[1], b_out[1][None, :], ln_g[1][None, :],
                  ln_b[1][None, :], norm_g[None, :], norm_b[None, :])
    return h
